# baseline (speedup 1.0000x reference)
import functools

import jax
import jax.numpy as jnp
from jax import lax
from jax.experimental import pallas as pl
from jax.experimental.pallas import tpu as pltpu
from jax.experimental.pallas import tpu_sc as plsc


def _proj_body(x_ref, w_ref, o_ref):
    o_ref[0] = lax.dot_general(
        x_ref[0], w_ref[...], (((0,), (1,)), ((), ())),
        preferred_element_type=jnp.float32)


def _matvec4(m_ref, b, i, p):
    t0 = m_ref[b, i, 0] * p[0]
    t1 = m_ref[b, i, 1] * p[1]
    t2 = m_ref[b, i, 2] * p[2]
    t3 = m_ref[b, i, 3] * p[3]
    return (t0 + t1) + (t2 + t3)


def _prep_body(pts_ref, bi_ref, tr_ref, r0_ref, p2_ref, wh_ref, idx_ref, w_ref,
               *, HF, WF, HW, V):
    x = pts_ref[0:1, :]
    y = pts_ref[1:2, :]
    z = pts_ref[2:3, :]
    one = pts_ref[3:4, :]
    bi = bi_ref[0:1, :]
    u = jnp.zeros_like(x)
    v = jnp.zeros_like(x)
    w = jnp.zeros_like(x)
    for b in range(4):
        p = (x, y, z, one)
        cam = tuple(_matvec4(tr_ref, b, i, p) for i in range(4))
        rect = tuple(_matvec4(r0_ref, b, i, cam) for i in range(4))
        ub = _matvec4(p2_ref, b, 0, rect)
        vb = _matvec4(p2_ref, b, 1, rect)
        wb = _matvec4(p2_ref, b, 2, rect)
        sel = bi == b
        u = jnp.where(sel, ub, u)
        v = jnp.where(sel, vb, v)
        w = jnp.where(sel, wb, w)
    depth = jnp.maximum(w, 1e-5)
    uu = u / depth
    vv = v / depth
    iw = wh_ref[0, 0]
    ih = wh_ref[0, 1]
    valid = (w > 0) & (uu >= 0.0) & (uu < iw) & (vv >= 0.0) & (vv < ih)
    validf = valid.astype(jnp.float32)
    x0 = jnp.floor(uu)
    y0 = jnp.floor(vv)
    fx = uu - x0
    fy = vv - y0
    wgt = ((1.0 - fx) * (1.0 - fy), fx * (1.0 - fy), (1.0 - fx) * fy, fx * fy)
    corners = ((0.0, 0.0), (1.0, 0.0), (0.0, 1.0), (1.0, 1.0))
    for c, (dx, dy) in enumerate(corners):
        xc = x0 + dx
        yc = y0 + dy
        inb = (xc >= 0.0) & (xc <= WF - 1.0) & (yc >= 0.0) & (yc <= HF - 1.0)
        wc = wgt[c] * inb.astype(jnp.float32) * validf
        xi = jnp.clip(xc, 0.0, WF - 1.0).astype(jnp.int32)
        yi = jnp.clip(yc, 0.0, HF - 1.0).astype(jnp.int32)
        flat = bi * HW + yi * WF + xi
        flat = jnp.clip(flat, 0, V - 1)
        idx_ref[c:c + 1, :] = flat
        w_ref[c:c + 1, :] = wc


def _make_fuse(N, PD, V, C):
    info = plsc.get_sparse_core_info()
    NC, NS = info.num_cores, info.num_subcores
    NW = NC * NS
    NCHUNK = N // C
    mesh = plsc.VectorSubcoreMesh(core_axis_name="c", subcore_axis_name="s")

    @functools.partial(
        pl.kernel, mesh=mesh,
        out_type=jax.ShapeDtypeStruct((N, PD), jnp.float32),
        scratch_types=[
            pltpu.VMEM((4 * C,), jnp.int32),
            pltpu.VMEM((4 * C,), jnp.int32),
            pltpu.VMEM((4 * C,), jnp.float32),
            pltpu.VMEM((4 * C,), jnp.float32),
            pltpu.VMEM((4, C, PD), jnp.float32),
            pltpu.VMEM((4, C, PD), jnp.float32),
            pltpu.VMEM((C, PD), jnp.float32),
            pltpu.VMEM((C, PD), jnp.float32),
            pltpu.VMEM((C, PD), jnp.float32),
            pltpu.VMEM((C, PD), jnp.float32),
            pltpu.VMEM((PD,), jnp.float32),
            pltpu.SemaphoreType.DMA,
            pltpu.SemaphoreType.DMA,
            pltpu.SemaphoreType.DMA,
            pltpu.SemaphoreType.DMA,
        ],
    )
    def fuse(pf_hbm, idx_hbm, w_hbm, pfeat_hbm, b_hbm, out_hbm,
             idxva, idxvb, wva, wvb, rowsa, rowsb, pfva, pfvb, outva, outvb,
             bv, sem_i, sem_g, sem_w, sem_o):
        idxs = (idxva, idxvb)
        wvs = (wva, wvb)
        rows = (rowsa, rowsb)
        pfvs = (pfva, pfvb)
        outs = (outva, outvb)
        wid = lax.axis_index("s") * NC + lax.axis_index("c")
        lo = wid * NCHUNK // NW
        hi = (wid + 1) * NCHUNK // NW
        cnt = hi - lo
        pltpu.sync_copy(b_hbm, bv)

        def issue_gathers(k, s):
            for c in range(4):
                pltpu.async_copy(pf_hbm.at[idxs[s].at[pl.ds(c * C, C)]],
                                 rows[s].at[c], sem_g)
            pltpu.async_copy(pfeat_hbm.at[pl.ds(k * C, C)], pfvs[s], sem_g)

        def drain_gathers(s):
            for c in range(4):
                pltpu.make_async_copy(pf_hbm.at[pl.ds(0, C)],
                                      rows[s].at[c], sem_g).wait()
            pltpu.make_async_copy(pfeat_hbm.at[pl.ds(0, C)], pfvs[s],
                                  sem_g).wait()

        pltpu.async_copy(idx_hbm.at[lo], idxva, sem_i)
        pltpu.make_async_copy(idx_hbm.at[0], idxva, sem_i).wait()
        issue_gathers(lo, 0)
        pltpu.async_copy(w_hbm.at[lo], wva, sem_w)

        @pl.when(cnt > 1)
        def _():
            pltpu.async_copy(idx_hbm.at[lo + 1], idxvb, sem_i)

        def pair_body(q, carry):
            for s in range(2):
                j = q * 2 + s
                k = lo + j
                s1 = 1 - s

                @pl.when(j < cnt)
                def _():
                    drain_gathers(s)

                    @pl.when(k + 2 < hi)
                    def _():
                        pltpu.async_copy(idx_hbm.at[k + 2], idxs[s], sem_i)

                    @pl.when(k + 1 < hi)
                    def _():
                        pltpu.make_async_copy(idx_hbm.at[0], idxs[s1],
                                              sem_i).wait()
                        issue_gathers(k + 1, s1)
                        pltpu.async_copy(w_hbm.at[k + 1], wvs[s1], sem_w)

                    pltpu.make_async_copy(w_hbm.at[0], wvs[s], sem_w).wait()

                    @pl.when(j >= 2)
                    def _():
                        pltpu.make_async_copy(outs[s], out_hbm.at[pl.ds(0, C)],
                                              sem_o).wait()

                    def group_body(g, carry2):
                        p0 = g * 16
                        wvecs = [wvs[s][pl.ds(c * C + p0, 16)] for c in range(4)]
                        pfg = pfvs[s].at[pl.ds(p0, 16)]
                        outg = outs[s].at[pl.ds(p0, 16)]
                        rg = [rows[s].at[c, pl.ds(p0, 16)] for c in range(4)]
                        for jj in range(16):
                            ws = [lax.broadcast(wvecs[c][jj], (16,))
                                  for c in range(4)]
                            for f in range(PD // 16):
                                sl = pl.ds(f * 16, 16)
                                acc = pfg[jj, sl] + bv[sl]
                                for c in range(4):
                                    acc = acc + ws[c] * rg[c][jj, sl]
                                outg[jj, sl] = acc
                        return carry2

                    lax.fori_loop(0, C // 16, group_body, 0)
                    pltpu.async_copy(outs[s], out_hbm.at[pl.ds(k * C, C)],
                                     sem_o)

            return carry

        lax.fori_loop(0, (cnt + 1) // 2, pair_body, 0)

        @pl.when(cnt >= 1)
        def _():
            pltpu.make_async_copy(outva, out_hbm.at[pl.ds(0, C)], sem_o).wait()

        @pl.when(cnt >= 2)
        def _():
            pltpu.make_async_copy(outva, out_hbm.at[pl.ds(0, C)], sem_o).wait()

    return fuse


def kernel(point_feat, pillar_centers, batch_idx, img_feat, P2, R0_rect,
           Tr_velo_to_cam, W_align, b_align, img_h, img_w):
    N, PD = point_feat.shape
    BS, C_IMG, HF, WF = img_feat.shape
    HW = HF * WF
    V = BS * HW

    TJ = 1280
    img_r = img_feat.reshape(BS, C_IMG, HW)
    pf = pl.pallas_call(
        _proj_body,
        grid=(BS, HW // TJ),
        in_specs=[
            pl.BlockSpec((1, C_IMG, TJ), lambda b, t: (b, 0, t)),
            pl.BlockSpec((PD, C_IMG), lambda b, t: (0, 0)),
        ],
        out_specs=pl.BlockSpec((1, TJ, PD), lambda b, t: (b, t, 0)),
        out_shape=jax.ShapeDtypeStruct((BS, HW, PD), jnp.float32),
    )(img_r, W_align)
    pf_flat = pf.reshape(V, PD)

    NT = 12800
    Npad = ((N + NT - 1) // NT) * NT
    pts4 = jnp.zeros((8, Npad), jnp.float32)
    pts4 = pts4.at[:3, :N].set(pillar_centers.T)
    pts4 = pts4.at[3, :N].set(1.0)
    bi2 = jnp.zeros((1, Npad), jnp.int32).at[0, :N].set(batch_idx.astype(jnp.int32))
    wh = jnp.stack([jnp.asarray(img_w), jnp.asarray(img_h)]).astype(jnp.float32).reshape(1, 2)

    idx4, w4 = pl.pallas_call(
        functools.partial(_prep_body, HF=HF, WF=WF, HW=HW, V=V),
        grid=(Npad // NT,),
        in_specs=[
            pl.BlockSpec((8, NT), lambda t: (0, t)),
            pl.BlockSpec((1, NT), lambda t: (0, t)),
            pl.BlockSpec(memory_space=pltpu.SMEM),
            pl.BlockSpec(memory_space=pltpu.SMEM),
            pl.BlockSpec(memory_space=pltpu.SMEM),
            pl.BlockSpec(memory_space=pltpu.SMEM),
        ],
        out_specs=[
            pl.BlockSpec((4, NT), lambda t: (0, t)),
            pl.BlockSpec((4, NT), lambda t: (0, t)),
        ],
        out_shape=[
            jax.ShapeDtypeStruct((4, Npad), jnp.int32),
            jax.ShapeDtypeStruct((4, Npad), jnp.float32),
        ],
    )(pts4, bi2, Tr_velo_to_cam, R0_rect, P2, wh)

    C = 64
    NCH = Npad // C
    idxt = jnp.transpose(idx4.reshape(4, NCH, C), (1, 0, 2)).reshape(NCH, 4 * C)
    wt = jnp.transpose(w4.reshape(4, NCH, C), (1, 0, 2)).reshape(NCH, 4 * C)
    fuse = _make_fuse(N, PD, V, C=C)
    out = fuse(pf_flat, idxt, wt, point_feat, b_align)
    return out


# speedup vs baseline: 1.6994x; 1.6994x over previous
import functools

import jax
import jax.numpy as jnp
from jax import lax
from jax.experimental import pallas as pl
from jax.experimental.pallas import tpu as pltpu
from jax.experimental.pallas import tpu_sc as plsc


def _proj_body(x_ref, w_ref, o_ref):
    o_ref[0] = lax.dot_general(
        x_ref[0], w_ref[...], (((0,), (1,)), ((), ())),
        preferred_element_type=jnp.float32)


def _matvec4(m_ref, b, i, p):
    t0 = m_ref[b, i, 0] * p[0]
    t1 = m_ref[b, i, 1] * p[1]
    t2 = m_ref[b, i, 2] * p[2]
    t3 = m_ref[b, i, 3] * p[3]
    return (t0 + t1) + (t2 + t3)


def _prep_body(pts_ref, bi_ref, tr_ref, r0_ref, p2_ref, wh_ref, idx_ref, w_ref,
               *, HF, WF, HW, V):
    x = pts_ref[0:1, :]
    y = pts_ref[1:2, :]
    z = pts_ref[2:3, :]
    one = pts_ref[3:4, :]
    bi = bi_ref[0:1, :]
    u = jnp.zeros_like(x)
    v = jnp.zeros_like(x)
    w = jnp.zeros_like(x)
    for b in range(4):
        p = (x, y, z, one)
        cam = tuple(_matvec4(tr_ref, b, i, p) for i in range(4))
        rect = tuple(_matvec4(r0_ref, b, i, cam) for i in range(4))
        ub = _matvec4(p2_ref, b, 0, rect)
        vb = _matvec4(p2_ref, b, 1, rect)
        wb = _matvec4(p2_ref, b, 2, rect)
        sel = bi == b
        u = jnp.where(sel, ub, u)
        v = jnp.where(sel, vb, v)
        w = jnp.where(sel, wb, w)
    depth = jnp.maximum(w, 1e-5)
    uu = u / depth
    vv = v / depth
    iw = wh_ref[0, 0]
    ih = wh_ref[0, 1]
    valid = (w > 0) & (uu >= 0.0) & (uu < iw) & (vv >= 0.0) & (vv < ih)
    validf = valid.astype(jnp.float32)
    x0 = jnp.floor(uu)
    y0 = jnp.floor(vv)
    fx = uu - x0
    fy = vv - y0
    wgt = ((1.0 - fx) * (1.0 - fy), fx * (1.0 - fy), (1.0 - fx) * fy, fx * fy)
    corners = ((0.0, 0.0), (1.0, 0.0), (0.0, 1.0), (1.0, 1.0))
    wcs = []
    for c, (dx, dy) in enumerate(corners):
        xc = x0 + dx
        yc = y0 + dy
        inb = (xc >= 0.0) & (xc <= WF - 1.0) & (yc >= 0.0) & (yc <= HF - 1.0)
        wcs.append(wgt[c] * inb.astype(jnp.float32) * validf)
    x0ge0 = x0 >= 0.0
    pxi = jnp.clip(x0, 0.0, WF - 1.0).astype(jnp.int32)
    zero = jnp.zeros_like(validf)
    for r, yc in enumerate((y0, y0 + 1.0)):
        yi = jnp.clip(yc, 0.0, HF - 1.0).astype(jnp.int32)
        flat = bi * HW + yi * WF + pxi
        flat = jnp.clip(flat, 0, V - 1)
        idx_ref[r:r + 1, :] = flat
        w_ref[2 * r:2 * r + 1, :] = jnp.where(x0ge0, wcs[2 * r], wcs[2 * r + 1])
        w_ref[2 * r + 1:2 * r + 2, :] = jnp.where(x0ge0, wcs[2 * r + 1], zero)


def _make_fuse(N, PD, V, C):
    info = plsc.get_sparse_core_info()
    NC, NS = info.num_cores, info.num_subcores
    NW = NC * NS
    NCHUNK = N // C
    mesh = plsc.VectorSubcoreMesh(core_axis_name="c", subcore_axis_name="s")

    @functools.partial(
        pl.kernel, mesh=mesh,
        out_type=jax.ShapeDtypeStruct((N, PD), jnp.float32),
        scratch_types=[
            pltpu.VMEM((2 * C,), jnp.int32),
            pltpu.VMEM((2 * C,), jnp.int32),
            pltpu.VMEM((4 * C,), jnp.float32),
            pltpu.VMEM((4 * C,), jnp.float32),
            pltpu.VMEM((2, C, 2 * PD), jnp.float32),
            pltpu.VMEM((2, C, 2 * PD), jnp.float32),
            pltpu.VMEM((C, PD), jnp.float32),
            pltpu.VMEM((C, PD), jnp.float32),
            pltpu.VMEM((C, PD), jnp.float32),
            pltpu.VMEM((C, PD), jnp.float32),
            pltpu.VMEM((PD,), jnp.float32),
            pltpu.SemaphoreType.DMA,
            pltpu.SemaphoreType.DMA,
            pltpu.SemaphoreType.DMA,
            pltpu.SemaphoreType.DMA,
        ],
    )
    def fuse(pf_hbm, idx_hbm, w_hbm, pfeat_hbm, b_hbm, out_hbm,
             idxva, idxvb, wva, wvb, rowsa, rowsb, pfva, pfvb, outva, outvb,
             bv, sem_i, sem_g, sem_w, sem_o):
        idxs = (idxva, idxvb)
        wvs = (wva, wvb)
        rows = (rowsa, rowsb)
        pfvs = (pfva, pfvb)
        outs = (outva, outvb)
        wid = lax.axis_index("s") * NC + lax.axis_index("c")
        lo = wid * NCHUNK // NW
        hi = (wid + 1) * NCHUNK // NW
        cnt = hi - lo
        pltpu.sync_copy(b_hbm, bv)

        def issue_gathers(k, s):
            for r in range(2):
                pltpu.async_copy(pf_hbm.at[idxs[s].at[pl.ds(r * C, C)]],
                                 rows[s].at[r], sem_g)
            pltpu.async_copy(pfeat_hbm.at[pl.ds(k * C, C)], pfvs[s], sem_g)

        def drain_gathers(s):
            for r in range(2):
                pltpu.make_async_copy(pf_hbm.at[pl.ds(0, C)],
                                      rows[s].at[r], sem_g).wait()
            pltpu.make_async_copy(pfeat_hbm.at[pl.ds(0, C)], pfvs[s],
                                  sem_g).wait()

        pltpu.async_copy(idx_hbm.at[lo], idxva, sem_i)
        pltpu.make_async_copy(idx_hbm.at[0], idxva, sem_i).wait()
        issue_gathers(lo, 0)
        pltpu.async_copy(w_hbm.at[lo], wva, sem_w)

        @pl.when(cnt > 1)
        def _():
            pltpu.async_copy(idx_hbm.at[lo + 1], idxvb, sem_i)

        def pair_body(q, carry):
            for s in range(2):
                j = q * 2 + s
                k = lo + j
                s1 = 1 - s

                @pl.when(j < cnt)
                def _():
                    drain_gathers(s)

                    @pl.when(k + 2 < hi)
                    def _():
                        pltpu.async_copy(idx_hbm.at[k + 2], idxs[s], sem_i)

                    @pl.when(k + 1 < hi)
                    def _():
                        pltpu.make_async_copy(idx_hbm.at[0], idxs[s1],
                                              sem_i).wait()
                        issue_gathers(k + 1, s1)
                        pltpu.async_copy(w_hbm.at[k + 1], wvs[s1], sem_w)

                    pltpu.make_async_copy(w_hbm.at[0], wvs[s], sem_w).wait()

                    @pl.when(j >= 2)
                    def _():
                        pltpu.make_async_copy(outs[s], out_hbm.at[pl.ds(0, C)],
                                              sem_o).wait()

                    def group_body(g, carry2):
                        p0 = g * 16
                        wvecs = [wvs[s][pl.ds(q * C + p0, 16)] for q in range(4)]
                        pfg = pfvs[s].at[pl.ds(p0, 16)]
                        outg = outs[s].at[pl.ds(p0, 16)]
                        rg = [rows[s].at[r, pl.ds(p0, 16)] for r in range(2)]
                        for jj in range(16):
                            ws = [lax.broadcast(wvecs[q][jj], (16,))
                                  for q in range(4)]
                            for f in range(PD // 16):
                                sl = pl.ds(f * 16, 16)
                                acc = pfg[jj, sl] + bv[sl]
                                for r in range(2):
                                    for a in range(2):
                                        v = rg[r][jj, pl.ds(a * PD + f * 16,
                                                            16)]
                                        acc = acc + ws[2 * r + a] * v
                                outg[jj, sl] = acc
                        return carry2

                    lax.fori_loop(0, C // 16, group_body, 0)
                    pltpu.async_copy(outs[s], out_hbm.at[pl.ds(k * C, C)],
                                     sem_o)

            return carry

        lax.fori_loop(0, (cnt + 1) // 2, pair_body, 0)

        @pl.when(cnt >= 1)
        def _():
            pltpu.make_async_copy(outva, out_hbm.at[pl.ds(0, C)], sem_o).wait()

        @pl.when(cnt >= 2)
        def _():
            pltpu.make_async_copy(outva, out_hbm.at[pl.ds(0, C)], sem_o).wait()

    return fuse


def kernel(point_feat, pillar_centers, batch_idx, img_feat, P2, R0_rect,
           Tr_velo_to_cam, W_align, b_align, img_h, img_w):
    N, PD = point_feat.shape
    BS, C_IMG, HF, WF = img_feat.shape
    HW = HF * WF
    V = BS * HW

    TJ = 1280
    img_r = img_feat.reshape(BS, C_IMG, HW)
    pf = pl.pallas_call(
        _proj_body,
        grid=(BS, HW // TJ),
        in_specs=[
            pl.BlockSpec((1, C_IMG, TJ), lambda b, t: (b, 0, t)),
            pl.BlockSpec((PD, C_IMG), lambda b, t: (0, 0)),
        ],
        out_specs=pl.BlockSpec((1, TJ, PD), lambda b, t: (b, t, 0)),
        out_shape=jax.ShapeDtypeStruct((BS, HW, PD), jnp.float32),
    )(img_r, W_align)
    pf_flat = pf.reshape(V, PD)
    pf2 = jnp.concatenate(
        [pf_flat, jnp.roll(pf_flat, -1, axis=0)], axis=1)

    NT = 12800
    Npad = ((N + NT - 1) // NT) * NT
    pts4 = jnp.zeros((8, Npad), jnp.float32)
    pts4 = pts4.at[:3, :N].set(pillar_centers.T)
    pts4 = pts4.at[3, :N].set(1.0)
    bi2 = jnp.zeros((1, Npad), jnp.int32).at[0, :N].set(batch_idx.astype(jnp.int32))
    wh = jnp.stack([jnp.asarray(img_w), jnp.asarray(img_h)]).astype(jnp.float32).reshape(1, 2)

    idx4, w4 = pl.pallas_call(
        functools.partial(_prep_body, HF=HF, WF=WF, HW=HW, V=V),
        grid=(Npad // NT,),
        in_specs=[
            pl.BlockSpec((8, NT), lambda t: (0, t)),
            pl.BlockSpec((1, NT), lambda t: (0, t)),
            pl.BlockSpec(memory_space=pltpu.SMEM),
            pl.BlockSpec(memory_space=pltpu.SMEM),
            pl.BlockSpec(memory_space=pltpu.SMEM),
            pl.BlockSpec(memory_space=pltpu.SMEM),
        ],
        out_specs=[
            pl.BlockSpec((2, NT), lambda t: (0, t)),
            pl.BlockSpec((4, NT), lambda t: (0, t)),
        ],
        out_shape=[
            jax.ShapeDtypeStruct((2, Npad), jnp.int32),
            jax.ShapeDtypeStruct((4, Npad), jnp.float32),
        ],
    )(pts4, bi2, Tr_velo_to_cam, R0_rect, P2, wh)

    C = 64
    NCH = Npad // C
    idxt = jnp.transpose(idx4.reshape(2, NCH, C), (1, 0, 2)).reshape(NCH, 2 * C)
    wt = jnp.transpose(w4.reshape(4, NCH, C), (1, 0, 2)).reshape(NCH, 4 * C)
    fuse = _make_fuse(N, PD, V, C=C)
    out = fuse(pf2, idxt, wt, point_feat, b_align)
    return out
